# Initial kernel scaffold; baseline (speedup 1.0000x reference)
import jax
import jax.numpy as jnp
from jax.experimental import pallas as pl


def kernel(x, y):
    raise NotImplementedError("write your pallas kernel here")


# speedup vs baseline: 1.0130x; 1.0130x over previous
import jax
import jax.numpy as jnp
from jax.experimental import pallas as pl
from jax.experimental.pallas import tpu as pltpu

_B = 16384
_V = 1000
_K = 11468
_R = 512
_G = _B // _R


def _ohem_body(x_ref, y_ref, o_ref, loss_sc):
    i = pl.program_id(0)
    x = x_ref[...]
    xm = jnp.max(x, axis=1, keepdims=True)
    s = jnp.sum(jnp.exp(x - xm), axis=1, keepdims=True)
    col = jax.lax.broadcasted_iota(jnp.int32, (_R, _V), 1)
    y = y_ref[...]
    xy = jnp.sum(jnp.where(col == y, x, 0.0), axis=1, keepdims=True)
    loss = (xm - xy) + jnp.log(s)
    lane = jax.lax.broadcasted_iota(jnp.int32, (_R, _G), 1)
    loss_sc[...] = jnp.where(lane == i, loss, loss_sc[...])

    @pl.when(i == _G - 1)
    def _select():
        vals = loss_sc[...]
        bits = jax.lax.bitcast_convert_type(vals, jnp.int32)

        def body(j, p):
            t = p | (jnp.int32(1) << (jnp.int32(30) - j))
            c = jnp.sum((bits >= t).astype(jnp.int32))
            return jnp.where(c >= _K, t, p)

        p = jax.lax.fori_loop(0, 31, body, jnp.int32(0))
        gt = bits > p
        c_gt = jnp.sum(gt.astype(jnp.int32))
        s_gt = jnp.sum(jnp.where(gt, vals, 0.0))
        tval = jnp.max(jnp.where(bits == p, vals, 0.0))
        total = s_gt + (jnp.int32(_K) - c_gt).astype(jnp.float32) * tval
        o_ref[0, 0] = total / jnp.float32(_K)


def kernel(x, y):
    y2 = y.astype(jnp.int32).reshape(_B, 1)
    out = pl.pallas_call(
        _ohem_body,
        grid=(_G,),
        in_specs=[
            pl.BlockSpec((_R, _V), lambda i: (i, 0)),
            pl.BlockSpec((_R, 1), lambda i: (i, 0)),
        ],
        out_specs=pl.BlockSpec(memory_space=pltpu.SMEM),
        out_shape=jax.ShapeDtypeStruct((1, 1), jnp.float32),
        scratch_shapes=[pltpu.VMEM((_R, _G), jnp.float32)],
        compiler_params=pltpu.CompilerParams(dimension_semantics=("arbitrary",)),
    )(x, y2)
    return out.reshape(())
